# baseline (speedup 1.0000x reference)
import jax
import jax.numpy as jnp
from jax.experimental import pallas as pl
from jax.experimental.pallas import tpu as pltpu

_BM = 80


def _fused(eps1_ref, eps2_ref, g1_ref, g2_ref, x1_ref, x2_ref,
           w10_ref, b10_ref, w11_ref, b11_ref, w12_ref, b12_ref,
           w20_ref, b20_ref, w21_ref, b21_ref, w22_ref, b22_ref,
           wm1_ref, wm2_ref, bm_ref, out_ref, x1b_ref, x2b_ref):
    @pl.when(pl.program_id(0) == 0)
    def _cast_x():
        x1b_ref[...] = x1_ref[...].astype(jnp.bfloat16)
        x2b_ref[...] = x2_ref[...].astype(jnp.bfloat16)

    agg1 = jnp.dot(g1_ref[...].astype(jnp.bfloat16), x1b_ref[...],
                   preferred_element_type=jnp.float32)
    h1 = (2.0 + eps1_ref[0, 0]) * agg1
    h1 = jnp.maximum(jnp.dot(h1, w10_ref[...]) + b10_ref[...], 0.0)
    h1 = jnp.maximum(jnp.dot(h1, w11_ref[...]) + b11_ref[...], 0.0)
    h1 = jnp.maximum(jnp.dot(h1, w12_ref[...]) + b12_ref[...], 0.0)

    agg2 = jnp.dot(g2_ref[...].astype(jnp.bfloat16), x2b_ref[...],
                   preferred_element_type=jnp.float32)
    h2 = (2.0 + eps2_ref[0, 0]) * agg2
    h2 = jnp.maximum(jnp.dot(h2, w20_ref[...]) + b20_ref[...], 0.0)
    h2 = jnp.maximum(jnp.dot(h2, w21_ref[...]) + b21_ref[...], 0.0)
    h2 = jnp.maximum(jnp.dot(h2, w22_ref[...]) + b22_ref[...], 0.0)

    out_ref[...] = (jnp.dot(h1, wm1_ref[...]) + jnp.dot(h2, wm2_ref[...])
                    + bm_ref[...])


def kernel(x1, x2, G1, G2, eps1, eps2, W10, b10, W11, b11, W12, b12,
           W20, b20, W21, b21, W22, b22, Wm, bm):
    n, d1 = x1.shape
    d2 = x2.shape[1]
    out = Wm.shape[1]

    eps1_2d = eps1.reshape(1, 1)
    eps2_2d = eps2.reshape(1, 1)
    wm1 = Wm[:out, :]
    wm2 = Wm[out:, :]

    def row_block(i):
        return (i, 0)

    def whole(i):
        return (0, 0)

    full = lambda shape: pl.BlockSpec(shape, whole)

    return pl.pallas_call(
        _fused,
        grid=(n // _BM,),
        in_specs=[
            full((1, 1)),
            full((1, 1)),
            pl.BlockSpec((_BM, n), row_block),
            pl.BlockSpec((_BM, n), row_block),
            full((n, d1)),
            full((n, d2)),
            full(W10.shape), full((1, out)),
            full(W11.shape), full((1, out)),
            full(W12.shape), full((1, out)),
            full(W20.shape), full((1, out)),
            full(W21.shape), full((1, out)),
            full(W22.shape), full((1, out)),
            full(wm1.shape), full(wm2.shape), full((1, out)),
        ],
        out_specs=pl.BlockSpec((_BM, out), row_block),
        out_shape=jax.ShapeDtypeStruct((n, out), jnp.float32),
        scratch_shapes=[
            pltpu.VMEM((n, d1), jnp.bfloat16),
            pltpu.VMEM((n, d2), jnp.bfloat16),
        ],
    )(eps1_2d, eps2_2d, G1, G2, x1, x2,
      W10, b10.reshape(1, -1), W11, b11.reshape(1, -1), W12, b12.reshape(1, -1),
      W20, b20.reshape(1, -1), W21, b21.reshape(1, -1), W22, b22.reshape(1, -1),
      wm1, wm2, bm.reshape(1, -1))


# speedup vs baseline: 1.1604x; 1.1604x over previous
import jax
import jax.numpy as jnp
from jax.experimental import pallas as pl

_BM = 320


def _fused(eps1_ref, eps2_ref, g1_ref, g2_ref, x1_ref, x2_ref,
           w10_ref, b10_ref, w11_ref, b11_ref, w12_ref, b12_ref,
           w20_ref, b20_ref, w21_ref, b21_ref, w22_ref, b22_ref,
           wm1_ref, wm2_ref, bm_ref, out_ref):
    agg1 = jnp.dot(g1_ref[...].astype(jnp.bfloat16), x1_ref[...],
                   preferred_element_type=jnp.float32)
    h1 = (2.0 + eps1_ref[0, 0]) * agg1
    h1 = jnp.maximum(jnp.dot(h1, w10_ref[...]) + b10_ref[...], 0.0)
    h1 = jnp.maximum(jnp.dot(h1, w11_ref[...]) + b11_ref[...], 0.0)
    h1 = jnp.maximum(jnp.dot(h1, w12_ref[...]) + b12_ref[...], 0.0)

    agg2 = jnp.dot(g2_ref[...].astype(jnp.bfloat16), x2_ref[...],
                   preferred_element_type=jnp.float32)
    h2 = (2.0 + eps2_ref[0, 0]) * agg2
    h2 = jnp.maximum(jnp.dot(h2, w20_ref[...]) + b20_ref[...], 0.0)
    h2 = jnp.maximum(jnp.dot(h2, w21_ref[...]) + b21_ref[...], 0.0)
    h2 = jnp.maximum(jnp.dot(h2, w22_ref[...]) + b22_ref[...], 0.0)

    out_ref[...] = (jnp.dot(h1, wm1_ref[...]) + jnp.dot(h2, wm2_ref[...])
                    + bm_ref[...])


def kernel(x1, x2, G1, G2, eps1, eps2, W10, b10, W11, b11, W12, b12,
           W20, b20, W21, b21, W22, b22, Wm, bm):
    n, d1 = x1.shape
    d2 = x2.shape[1]
    out = Wm.shape[1]

    eps1_2d = eps1.reshape(1, 1)
    eps2_2d = eps2.reshape(1, 1)
    wm1 = Wm[:out, :]
    wm2 = Wm[out:, :]

    def row_block(i):
        return (i, 0)

    def whole(i):
        return (0, 0)

    full = lambda shape: pl.BlockSpec(shape, whole)

    return pl.pallas_call(
        _fused,
        grid=(pl.cdiv(n, _BM),),
        in_specs=[
            full((1, 1)),
            full((1, 1)),
            pl.BlockSpec((_BM, n), row_block),
            pl.BlockSpec((_BM, n), row_block),
            full((n, d1)),
            full((n, d2)),
            full(W10.shape), full((1, out)),
            full(W11.shape), full((1, out)),
            full(W12.shape), full((1, out)),
            full(W20.shape), full((1, out)),
            full(W21.shape), full((1, out)),
            full(W22.shape), full((1, out)),
            full(wm1.shape), full(wm2.shape), full((1, out)),
        ],
        out_specs=pl.BlockSpec((_BM, out), row_block),
        out_shape=jax.ShapeDtypeStruct((n, out), jnp.float32),
    )(eps1_2d, eps2_2d, G1, G2,
      x1.astype(jnp.bfloat16), x2.astype(jnp.bfloat16),
      W10, b10.reshape(1, -1), W11, b11.reshape(1, -1), W12, b12.reshape(1, -1),
      W20, b20.reshape(1, -1), W21, b21.reshape(1, -1), W22, b22.reshape(1, -1),
      wm1, wm2, bm.reshape(1, -1))


# speedup vs baseline: 1.2844x; 1.1068x over previous
import jax
import jax.numpy as jnp
from jax.experimental import pallas as pl
from jax.experimental.pallas import tpu as pltpu

_BM = 200


def _fused(eps1_ref, eps2_ref, g1_ref, g2_ref, x1_ref, x2_ref,
           w10_ref, b10_ref, w11_ref, b11_ref, w12_ref, b12_ref,
           w20_ref, b20_ref, w21_ref, b21_ref, w22_ref, b22_ref,
           wm1_ref, wm2_ref, bm_ref, out_ref, x1b_ref, x2b_ref):
    @pl.when(pl.program_id(0) == 0)
    def _cast_x():
        x1b_ref[...] = x1_ref[...].astype(jnp.bfloat16)
        x2b_ref[...] = x2_ref[...].astype(jnp.bfloat16)

    agg1 = jnp.dot(g1_ref[...].astype(jnp.bfloat16), x1b_ref[...],
                   preferred_element_type=jnp.float32)
    agg2 = jnp.dot(g2_ref[...].astype(jnp.bfloat16), x2b_ref[...],
                   preferred_element_type=jnp.float32)
    h1 = (2.0 + eps1_ref[0, 0]) * agg1
    h1 = jnp.maximum(jnp.dot(h1, w10_ref[...]) + b10_ref[...], 0.0)
    h1 = jnp.maximum(jnp.dot(h1, w11_ref[...]) + b11_ref[...], 0.0)
    h1 = jnp.maximum(jnp.dot(h1, w12_ref[...]) + b12_ref[...], 0.0)
    h2 = (2.0 + eps2_ref[0, 0]) * agg2
    h2 = jnp.maximum(jnp.dot(h2, w20_ref[...]) + b20_ref[...], 0.0)
    h2 = jnp.maximum(jnp.dot(h2, w21_ref[...]) + b21_ref[...], 0.0)
    h2 = jnp.maximum(jnp.dot(h2, w22_ref[...]) + b22_ref[...], 0.0)
    out_ref[...] = (jnp.dot(h1, wm1_ref[...]) + jnp.dot(h2, wm2_ref[...])
                    + bm_ref[...])


def kernel(x1, x2, G1, G2, eps1, eps2, W10, b10, W11, b11, W12, b12,
           W20, b20, W21, b21, W22, b22, Wm, bm):
    n, d1 = x1.shape
    d2 = x2.shape[1]
    out = Wm.shape[1]

    eps1_2d = eps1.reshape(1, 1)
    eps2_2d = eps2.reshape(1, 1)
    wm1 = Wm[:out, :]
    wm2 = Wm[out:, :]

    def row_block(i):
        return (i, 0)

    def whole(i):
        return (0, 0)

    full = lambda shape: pl.BlockSpec(shape, whole)

    return pl.pallas_call(
        _fused,
        grid=(n // _BM,),
        in_specs=[
            full((1, 1)),
            full((1, 1)),
            pl.BlockSpec((_BM, n), row_block),
            pl.BlockSpec((_BM, n), row_block),
            full((n, d1)),
            full((n, d2)),
            full(W10.shape), full((1, out)),
            full(W11.shape), full((1, out)),
            full(W12.shape), full((1, out)),
            full(W20.shape), full((1, out)),
            full(W21.shape), full((1, out)),
            full(W22.shape), full((1, out)),
            full(wm1.shape), full(wm2.shape), full((1, out)),
        ],
        out_specs=pl.BlockSpec((_BM, out), row_block),
        out_shape=jax.ShapeDtypeStruct((n, out), jnp.float32),
        scratch_shapes=[
            pltpu.VMEM((n, d1), jnp.bfloat16),
            pltpu.VMEM((n, d2), jnp.bfloat16),
        ],
    )(eps1_2d, eps2_2d, G1, G2, x1, x2,
      W10, b10.reshape(1, -1), W11, b11.reshape(1, -1), W12, b12.reshape(1, -1),
      W20, b20.reshape(1, -1), W21, b21.reshape(1, -1), W22, b22.reshape(1, -1),
      wm1, wm2, bm.reshape(1, -1))
